# baseline (device time: 12260 ns/iter reference)
import jax
import jax.numpy as jnp
from jax import lax
from jax.experimental import pallas as pl
from jax.experimental.pallas import tpu as pltpu

M = 512
N_HALF = 512
M_HALF = 256
C = 4
R = M_HALF // C


def kernel(x):
    def body(
        x_ref,
        out_ref,
        sv,
        sbuf,
        yrecv,
        xrecv,
        kin,
        in_sems,
        keep_sem,
        ysend_sems,
        yrecv_sems,
        xsend_sems,
        xrecv_sems,
    ):
        my_x = lax.axis_index("x")
        my_y = lax.axis_index("y")
        peer_y = (my_x, 1 - my_y)
        peer_x = (1 - my_x, my_y)

        col_keep = my_y * N_HALF
        col_send = (1 - my_y) * N_HALF
        row0 = my_x * M_HALF
        other0 = (1 - my_x) * M_HALF

        barrier_sem = pltpu.get_barrier_semaphore()
        for nbr in (peer_y, peer_x):
            pl.semaphore_signal(
                barrier_sem,
                inc=1,
                device_id=nbr,
                device_id_type=pl.DeviceIdType.MESH,
            )

        in_dmas = []
        for c in range(C):
            dma = pltpu.make_async_copy(
                x_ref.at[0, pl.ds(row0 + c * R, R), pl.ds(col_send, N_HALF)],
                sv.at[pl.ds(c * R, R)],
                in_sems.at[c],
            )
            dma.start()
            in_dmas.append(dma)
        keep_dma = pltpu.make_async_copy(
            x_ref.at[0, :, pl.ds(col_keep, N_HALF)], kin, keep_sem
        )
        keep_dma.start()

        y_rdmas = []
        for c in range(C):
            sl = pl.ds(c * R, R)
            in_dmas[c].wait()
            sbuf[sl] = sv[sl].astype(jnp.bfloat16)
            if c == 0:
                pl.semaphore_wait(barrier_sem, 2)
            rdma = pltpu.make_async_remote_copy(
                src_ref=sbuf.at[sl],
                dst_ref=yrecv.at[sl],
                send_sem=ysend_sems.at[c],
                recv_sem=yrecv_sems.at[c],
                device_id=peer_y,
                device_id_type=pl.DeviceIdType.MESH,
            )
            rdma.start()
            y_rdmas.append(rdma)

        x_rdmas = []
        for c in range(C):
            sl = pl.ds(c * R, R)
            y_rdmas[c].wait_recv()
            fwd = pltpu.make_async_remote_copy(
                src_ref=yrecv.at[sl],
                dst_ref=xrecv.at[sl],
                send_sem=xsend_sems.at[c],
                recv_sem=xrecv_sems.at[c],
                device_id=peer_x,
                device_id_type=pl.DeviceIdType.MESH,
            )
            fwd.start()
            x_rdmas.append(fwd)

        keep_dma.wait()
        out_ref[pl.ds(row0, M_HALF), :] = (
            kin[pl.ds(row0, M_HALF)] + yrecv[...].astype(jnp.float32)
        ).astype(jnp.bfloat16)

        for c in range(C):
            x_rdmas[c].wait_recv()
        out_ref[pl.ds(other0, M_HALF), :] = (
            kin[pl.ds(other0, M_HALF)] + xrecv[...].astype(jnp.float32)
        ).astype(jnp.bfloat16)

        for c in range(C):
            y_rdmas[c].wait_send()
            x_rdmas[c].wait_send()

    return pl.pallas_call(
        body,
        out_shape=jax.ShapeDtypeStruct((M, N_HALF), jnp.bfloat16),
        in_specs=[pl.BlockSpec(memory_space=pl.ANY)],
        out_specs=pl.BlockSpec(memory_space=pltpu.VMEM),
        scratch_shapes=[
            pltpu.VMEM((M_HALF, N_HALF), jnp.float32),
            pltpu.VMEM((M_HALF, N_HALF), jnp.bfloat16),
            pltpu.VMEM((M_HALF, N_HALF), jnp.bfloat16),
            pltpu.VMEM((M_HALF, N_HALF), jnp.bfloat16),
            pltpu.VMEM((M, N_HALF), jnp.float32),
            pltpu.SemaphoreType.DMA((C,)),
            pltpu.SemaphoreType.DMA,
            pltpu.SemaphoreType.DMA((C,)),
            pltpu.SemaphoreType.DMA((C,)),
            pltpu.SemaphoreType.DMA((C,)),
            pltpu.SemaphoreType.DMA((C,)),
        ],
        compiler_params=pltpu.CompilerParams(collective_id=0),
    )(x)


# device time: 12138 ns/iter; 1.0101x vs baseline; 1.0101x over previous
import jax
import jax.numpy as jnp
from jax import lax
from jax.experimental import pallas as pl
from jax.experimental.pallas import tpu as pltpu

M = 512
N_HALF = 512
M_HALF = 256
C = 8
R = M_HALF // C


def kernel(x):
    def body(
        x_ref,
        out_ref,
        sv,
        sbuf,
        yrecv,
        xrecv,
        kin,
        in_sems,
        keep_sem,
        ysend_sems,
        yrecv_sems,
        xsend_sems,
        xrecv_sems,
    ):
        my_x = lax.axis_index("x")
        my_y = lax.axis_index("y")
        peer_y = (my_x, 1 - my_y)
        peer_x = (1 - my_x, my_y)

        col_keep = my_y * N_HALF
        col_send = (1 - my_y) * N_HALF
        row0 = my_x * M_HALF
        other0 = (1 - my_x) * M_HALF

        barrier_sem = pltpu.get_barrier_semaphore()
        for nbr in (peer_y, peer_x):
            pl.semaphore_signal(
                barrier_sem,
                inc=1,
                device_id=nbr,
                device_id_type=pl.DeviceIdType.MESH,
            )

        in_dmas = []
        for c in range(C):
            dma = pltpu.make_async_copy(
                x_ref.at[0, pl.ds(row0 + c * R, R), pl.ds(col_send, N_HALF)],
                sv.at[pl.ds(c * R, R)],
                in_sems.at[c],
            )
            dma.start()
            in_dmas.append(dma)
        keep_dma = pltpu.make_async_copy(
            x_ref.at[0, :, pl.ds(col_keep, N_HALF)], kin, keep_sem
        )
        keep_dma.start()

        y_rdmas = []
        for c in range(C):
            sl = pl.ds(c * R, R)
            in_dmas[c].wait()
            sbuf[sl] = sv[sl].astype(jnp.bfloat16)
            if c == 0:
                pl.semaphore_wait(barrier_sem, 2)
            rdma = pltpu.make_async_remote_copy(
                src_ref=sbuf.at[sl],
                dst_ref=yrecv.at[sl],
                send_sem=ysend_sems.at[c],
                recv_sem=yrecv_sems.at[c],
                device_id=peer_y,
                device_id_type=pl.DeviceIdType.MESH,
            )
            rdma.start()
            y_rdmas.append(rdma)

        x_rdmas = []
        for c in range(C):
            sl = pl.ds(c * R, R)
            y_rdmas[c].wait_recv()
            fwd = pltpu.make_async_remote_copy(
                src_ref=yrecv.at[sl],
                dst_ref=xrecv.at[sl],
                send_sem=xsend_sems.at[c],
                recv_sem=xrecv_sems.at[c],
                device_id=peer_x,
                device_id_type=pl.DeviceIdType.MESH,
            )
            fwd.start()
            x_rdmas.append(fwd)

        keep_dma.wait()
        out_ref[pl.ds(row0, M_HALF), :] = (
            kin[pl.ds(row0, M_HALF)] + yrecv[...].astype(jnp.float32)
        ).astype(jnp.bfloat16)

        for c in range(C):
            x_rdmas[c].wait_recv()
        out_ref[pl.ds(other0, M_HALF), :] = (
            kin[pl.ds(other0, M_HALF)] + xrecv[...].astype(jnp.float32)
        ).astype(jnp.bfloat16)

        for c in range(C):
            y_rdmas[c].wait_send()
            x_rdmas[c].wait_send()

    return pl.pallas_call(
        body,
        out_shape=jax.ShapeDtypeStruct((M, N_HALF), jnp.bfloat16),
        in_specs=[pl.BlockSpec(memory_space=pl.ANY)],
        out_specs=pl.BlockSpec(memory_space=pltpu.VMEM),
        scratch_shapes=[
            pltpu.VMEM((M_HALF, N_HALF), jnp.float32),
            pltpu.VMEM((M_HALF, N_HALF), jnp.bfloat16),
            pltpu.VMEM((M_HALF, N_HALF), jnp.bfloat16),
            pltpu.VMEM((M_HALF, N_HALF), jnp.bfloat16),
            pltpu.VMEM((M, N_HALF), jnp.float32),
            pltpu.SemaphoreType.DMA((C,)),
            pltpu.SemaphoreType.DMA,
            pltpu.SemaphoreType.DMA((C,)),
            pltpu.SemaphoreType.DMA((C,)),
            pltpu.SemaphoreType.DMA((C,)),
            pltpu.SemaphoreType.DMA((C,)),
        ],
        compiler_params=pltpu.CompilerParams(collective_id=0),
    )(x)
